# baseline (speedup 1.0000x reference)
import functools

import jax
import jax.numpy as jnp
from jax import lax
from jax.experimental import pallas as pl
from jax.experimental.pallas import tpu as pltpu
from jax.experimental.pallas import tpu_sc as plsc

VOCAB = 100000
EMBED = 1024
MAXLEN = 2048
BATCH = 4
SEQ = 2048

NUM_WORKERS = 32
S_PER_W = SEQ // NUM_WORKERS
ROWS_PER_W = BATCH * S_PER_W
CHUNK = 8
NGROUPS = S_PER_W // CHUNK
NBUF = 3 * BATCH
NPOS = 2
VPB = 16


def _emb_kernel(
    tok_hbm, ids_hbm, pos_hbm, out_hbm, idx_v, tok_v, pos_v, sem_i, sem_g,
    sem_p, sem_o
):
    nc = 2
    wid = lax.axis_index("s") * nc + lax.axis_index("c")
    s0 = wid * S_PER_W

    idx_cps = [
        pltpu.async_copy(
            ids_hbm.at[pl.ds(b * SEQ + s0, S_PER_W)],
            idx_v.at[pl.ds(b * S_PER_W, S_PER_W)],
            sem_i,
        )
        for b in range(BATCH)
    ]

    def start_pos(g):
        return pltpu.async_copy(
            pos_hbm.at[pl.ds(s0 + g * CHUNK, CHUNK)],
            pos_v.at[g % NPOS],
            sem_p.at[g % NPOS],
        )

    def start_gathers(g):
        cps = []
        for b in range(BATCH):
            slot = (g * BATCH + b) % NBUF
            cps.append(
                pltpu.async_copy(
                    tok_hbm.at[idx_v.at[pl.ds(b * S_PER_W + g * CHUNK, CHUNK)]],
                    tok_v.at[slot],
                    sem_g.at[slot],
                )
            )
        return cps

    def add_group(g):
        ps = g % NPOS
        slots = [(g * BATCH + b) % NBUF for b in range(BATCH)]

        def add_row(r):
            for grp in range(EMBED // (16 * VPB)):
                vs = [
                    pos_v[ps, r, pl.ds((grp * VPB + j) * 16, 16)]
                    for j in range(VPB)
                ]
                for j in range(VPB):
                    sl = pl.ds((grp * VPB + j) * 16, 16)
                    for slot in slots:
                        plsc.addupdate(tok_v.at[slot, r, sl], vs[j])

        pl.loop(0, CHUNK, unroll=2)(add_row)

    def start_outs(g):
        cps = []
        for b in range(BATCH):
            slot = (g * BATCH + b) % NBUF
            cps.append(
                pltpu.async_copy(
                    tok_v.at[slot],
                    out_hbm.at[pl.ds(b * SEQ + s0 + g * CHUNK, CHUNK)],
                    sem_o.at[slot],
                )
            )
        return cps

    pos_cps = {g: start_pos(g) for g in range(NPOS)}
    for cp in idx_cps:
        cp.wait()
    gather_cps = {0: start_gathers(0)}

    out_cps = {}
    for g in range(NGROUPS):
        if g >= 2:
            for cp in out_cps.pop(g - 2):
                cp.wait()
        if g + 1 < NGROUPS:
            gather_cps[g + 1] = start_gathers(g + 1)
        for cp in gather_cps.pop(g):
            cp.wait()
        pos_cps.pop(g).wait()
        add_group(g)
        out_cps[g] = start_outs(g)
        if g + NPOS < NGROUPS:
            pos_cps[g + NPOS] = start_pos(g + NPOS)
    for g in sorted(out_cps):
        for cp in out_cps.pop(g):
            cp.wait()


@jax.jit
def _embedding(ids_flat, token_table, pos_table):
    mesh = plsc.VectorSubcoreMesh(core_axis_name="c", subcore_axis_name="s")
    k = functools.partial(
        pl.kernel,
        mesh=mesh,
        out_type=jax.ShapeDtypeStruct((BATCH * SEQ, EMBED), jnp.float32),
        scratch_types=[
            pltpu.VMEM((ROWS_PER_W,), jnp.int32),
            pltpu.VMEM((NBUF, CHUNK, EMBED), jnp.float32),
            pltpu.VMEM((NPOS, CHUNK, EMBED), jnp.float32),
            pltpu.SemaphoreType.DMA,
            pltpu.SemaphoreType.DMA((NBUF,)),
            pltpu.SemaphoreType.DMA((NPOS,)),
            pltpu.SemaphoreType.DMA((NBUF,)),
        ],
    )(_emb_kernel)
    return k(token_table, ids_flat, pos_table)


def kernel(input_ids, token_table, pos_table):
    batch, seq = input_ids.shape
    ids_flat = input_ids.reshape(batch * seq).astype(jnp.int32)
    out = _embedding(ids_flat, token_table, pos_table)
    return out.reshape(batch, seq, EMBED)


# speedup vs baseline: 1.0019x; 1.0019x over previous
import functools

import jax
import jax.numpy as jnp
from jax import lax
from jax.experimental import pallas as pl
from jax.experimental.pallas import tpu as pltpu
from jax.experimental.pallas import tpu_sc as plsc

VOCAB = 100000
EMBED = 1024
MAXLEN = 2048
BATCH = 4
SEQ = 2048

NUM_WORKERS = 32
S_PER_W = SEQ // NUM_WORKERS
ROWS_PER_W = BATCH * S_PER_W
CHUNK = 8
NGROUPS = S_PER_W // CHUNK
NBUF = 3 * BATCH
NPOS = 2
VPB = 16


def _emb_kernel(
    tok_hbm, ids_hbm, pos_hbm, out_hbm, idx_v, tok_v, pos_v, sem_i, sem_g,
    sem_p, sem_o
):
    nc = 2
    wid = lax.axis_index("s") * nc + lax.axis_index("c")
    s0 = wid * S_PER_W

    idx_cps = [
        pltpu.async_copy(
            ids_hbm.at[pl.ds(b * SEQ + s0, S_PER_W)],
            idx_v.at[pl.ds(b * S_PER_W, S_PER_W)],
            sem_i,
        )
        for b in range(BATCH)
    ]

    def start_pos(g):
        return pltpu.async_copy(
            pos_hbm.at[pl.ds(s0 + g * CHUNK, CHUNK)],
            pos_v.at[g % NPOS],
            sem_p.at[g % NPOS],
        )

    def start_gathers(g):
        cps = []
        for b in range(BATCH):
            slot = (g * BATCH + b) % NBUF
            cps.append(
                pltpu.async_copy(
                    tok_hbm.at[idx_v.at[pl.ds(b * S_PER_W + g * CHUNK, CHUNK)]],
                    tok_v.at[slot],
                    sem_g.at[slot],
                )
            )
        return cps

    def add_group(g):
        ps = g % NPOS
        slots = [(g * BATCH + b) % NBUF for b in range(BATCH)]

        def add_row(r):
            for grp in range(EMBED // (16 * VPB)):
                vs = [
                    pos_v[ps, r, pl.ds((grp * VPB + j) * 16, 16)]
                    for j in range(VPB)
                ]
                for slot in slots:
                    for j in range(VPB):
                        sl = pl.ds((grp * VPB + j) * 16, 16)
                        plsc.addupdate(tok_v.at[slot, r, sl], vs[j])

        pl.loop(0, CHUNK, unroll=2)(add_row)

    def start_outs(g):
        cps = []
        for b in range(BATCH):
            slot = (g * BATCH + b) % NBUF
            cps.append(
                pltpu.async_copy(
                    tok_v.at[slot],
                    out_hbm.at[pl.ds(b * SEQ + s0 + g * CHUNK, CHUNK)],
                    sem_o.at[slot],
                )
            )
        return cps

    pos_cps = {g: start_pos(g) for g in range(NPOS)}
    for cp in idx_cps:
        cp.wait()
    gather_cps = {0: start_gathers(0)}

    out_cps = {}
    for g in range(NGROUPS):
        if g >= 2:
            for cp in out_cps.pop(g - 2):
                cp.wait()
        if g + 1 < NGROUPS:
            gather_cps[g + 1] = start_gathers(g + 1)
        for cp in gather_cps.pop(g):
            cp.wait()
        pos_cps.pop(g).wait()
        add_group(g)
        out_cps[g] = start_outs(g)
        if g + NPOS < NGROUPS:
            pos_cps[g + NPOS] = start_pos(g + NPOS)
    for g in sorted(out_cps):
        for cp in out_cps.pop(g):
            cp.wait()


@jax.jit
def _embedding(ids_flat, token_table, pos_table):
    mesh = plsc.VectorSubcoreMesh(core_axis_name="c", subcore_axis_name="s")
    k = functools.partial(
        pl.kernel,
        mesh=mesh,
        out_type=jax.ShapeDtypeStruct((BATCH * SEQ, EMBED), jnp.float32),
        scratch_types=[
            pltpu.VMEM((ROWS_PER_W,), jnp.int32),
            pltpu.VMEM((NBUF, CHUNK, EMBED), jnp.float32),
            pltpu.VMEM((NPOS, CHUNK, EMBED), jnp.float32),
            pltpu.SemaphoreType.DMA,
            pltpu.SemaphoreType.DMA((NBUF,)),
            pltpu.SemaphoreType.DMA((NPOS,)),
            pltpu.SemaphoreType.DMA((NBUF,)),
        ],
    )(_emb_kernel)
    return k(token_table, ids_flat, pos_table)


def kernel(input_ids, token_table, pos_table):
    batch, seq = input_ids.shape
    ids_flat = input_ids.reshape(batch * seq).astype(jnp.int32)
    out = _embedding(ids_flat, token_table, pos_table)
    return out.reshape(batch, seq, EMBED)


# speedup vs baseline: 1.0658x; 1.0637x over previous
import functools

import jax
import jax.numpy as jnp
from jax import lax
from jax.experimental import pallas as pl
from jax.experimental.pallas import tpu as pltpu
from jax.experimental.pallas import tpu_sc as plsc

VOCAB = 100000
EMBED = 1024
MAXLEN = 2048
BATCH = 4
SEQ = 2048

NUM_WORKERS = 32
S_PER_W = SEQ // NUM_WORKERS
ROWS_PER_W = BATCH * S_PER_W
CHUNK = 16
CHUNKS_PER_B = S_PER_W // CHUNK
NCHUNKS = BATCH * CHUNKS_PER_B
NBUF = 5
NPOS = 2
LAG = 2


def _emb_kernel(
    tok_hbm, ids_hbm, pos_hbm, out_hbm, idx_v, tok_v, pos_v, sem_i, sem_g,
    sem_p, sem_o
):
    nc = 2
    wid = lax.axis_index("s") * nc + lax.axis_index("c")
    s0 = wid * S_PER_W

    idx_cps = [
        pltpu.async_copy(
            ids_hbm.at[pl.ds(b * SEQ + s0, S_PER_W)],
            idx_v.at[pl.ds(b * S_PER_W, S_PER_W)],
            sem_i,
        )
        for b in range(BATCH)
    ]

    def start_pos(c):
        return pltpu.async_copy(
            pos_hbm.at[pl.ds(s0 + c * CHUNK, CHUNK)],
            pos_v.at[c % NPOS],
            sem_p.at[c % NPOS],
        )

    def start_chunk(i):
        c, b = divmod(i, BATCH)
        return pltpu.async_copy(
            tok_hbm.at[idx_v.at[pl.ds(b * S_PER_W + c * CHUNK, CHUNK)]],
            tok_v.at[i % NBUF],
            sem_g.at[i % NBUF],
        )

    def finish_chunk(i, g, pos_cps):
        c, b = divmod(i, BATCH)
        slot = i % NBUF
        if b == 0:
            with jax.named_scope("wait_pos"):
                pos_cps.pop(c).wait()
        with jax.named_scope("wait_gather"):
            g.wait()

        def add_row(r):
            for grp in range(EMBED // 512):
                vs = [
                    pos_v[c % NPOS, r, pl.ds((grp * 32 + j) * 16, 16)]
                    for j in range(32)
                ]
                for j in range(32):
                    plsc.addupdate(
                        tok_v.at[slot, r, pl.ds((grp * 32 + j) * 16, 16)],
                        vs[j],
                    )

        with jax.named_scope("add_pos"):
            pl.loop(0, CHUNK, unroll=2)(add_row)
        if b == BATCH - 1 and c + NPOS < CHUNKS_PER_B:
            pos_cps[c + NPOS] = start_pos(c + NPOS)
        return pltpu.async_copy(
            tok_v.at[slot],
            out_hbm.at[pl.ds(b * SEQ + s0 + c * CHUNK, CHUNK)],
            sem_o.at[slot],
        )

    pos_cps = {c: start_pos(c) for c in range(NPOS)}
    for cp in idx_cps:
        cp.wait()

    inflight = {}
    out_cp = {}
    for i in range(NCHUNKS + LAG):
        if i < NCHUNKS:
            if i >= NBUF:
                with jax.named_scope("wait_out"):
                    out_cp.pop(i - NBUF).wait()
            inflight[i] = start_chunk(i)
        j = i - LAG
        if j >= 0:
            out_cp[j] = finish_chunk(j, inflight.pop(j), pos_cps)
    for j in sorted(out_cp):
        out_cp.pop(j).wait()


@jax.jit
def _embedding(ids_flat, token_table, pos_table):
    mesh = plsc.VectorSubcoreMesh(core_axis_name="c", subcore_axis_name="s")
    k = functools.partial(
        pl.kernel,
        mesh=mesh,
        out_type=jax.ShapeDtypeStruct((BATCH * SEQ, EMBED), jnp.float32),
        scratch_types=[
            pltpu.VMEM((ROWS_PER_W,), jnp.int32),
            pltpu.VMEM((NBUF, CHUNK, EMBED), jnp.float32),
            pltpu.VMEM((NPOS, CHUNK, EMBED), jnp.float32),
            pltpu.SemaphoreType.DMA,
            pltpu.SemaphoreType.DMA((NBUF,)),
            pltpu.SemaphoreType.DMA((NPOS,)),
            pltpu.SemaphoreType.DMA((NBUF,)),
        ],
    )(_emb_kernel)
    return k(token_table, ids_flat, pos_table)


def kernel(input_ids, token_table, pos_table):
    batch, seq = input_ids.shape
    ids_flat = input_ids.reshape(batch * seq).astype(jnp.int32)
    out = _embedding(ids_flat, token_table, pos_table)
    return out.reshape(batch, seq, EMBED)
